# Initial kernel scaffold; baseline (speedup 1.0000x reference)
import jax
import jax.numpy as jnp
from jax.experimental import pallas as pl


def kernel(x, edge_index, edge_attr, W_self, W_edge, W_nbr, b, weights_list):
    raise NotImplementedError("write your pallas kernel here")


# speedup vs baseline: 2.6908x; 2.6908x over previous
import functools

import jax
import jax.numpy as jnp
from jax import lax
from jax.experimental import pallas as pl
from jax.experimental.pallas import tpu as pltpu
from jax.experimental.pallas import tpu_sc as plsc

_N = 10000
_E = 320000
_D = 128
_DE = 16
_NUM_BLOCKS = 2

_NC = 2
_NS = 16
_NW = _NC * _NS
_EPW = _E // _NW
_CH = 80
_NCH = _EPW // _CH
_ZR = 200
_NZC = _N // _ZR


_BE = 8000


def _emsg_body(ea_ref, we_ref, out_ref):
    out_ref[...] = jnp.dot(ea_ref[...], we_ref[...],
                           preferred_element_type=jnp.float32)


def _edge_messages(edge_attr, W_edge):
    return pl.pallas_call(
        _emsg_body,
        grid=(_E // _BE,),
        in_specs=[
            pl.BlockSpec((_BE, _DE), lambda i: (i, 0)),
            pl.BlockSpec((_DE, _D), lambda i: (0, 0)),
        ],
        out_specs=pl.BlockSpec((_BE, _D), lambda i: (i, 0)),
        out_shape=jax.ShapeDtypeStruct((_E, _D), jnp.float32),
    )(edge_attr, W_edge)


def _sc_body(x_hbm, src_hbm, dst_hbm, emsg_hbm, out_hbm,
             idxs_v, idxd_v, emsg_v, xrow_v, zbuf_v, agg_sh, sem):
    c = lax.axis_index("c")
    s = lax.axis_index("s")
    wid = c * _NS + s

    def _zfill(i, _):
        r = i // 8
        col = (i % 8) * 16
        zbuf_v[r, pl.ds(col, 16)] = jnp.zeros((16,), jnp.float32)
        return 0
    lax.fori_loop(0, _ZR * 8, _zfill, 0)
    my_nz = jnp.where(s < _NZC - (_NZC // _NS) * _NS, _NZC // _NS + 1,
                      _NZC // _NS)

    def _zchunk(i, _):
        r = (s + i * _NS) * _ZR
        pltpu.sync_copy(zbuf_v, agg_sh.at[pl.ds(r, _ZR)])
        return 0
    lax.fori_loop(0, my_nz, _zchunk, 0)
    plsc.subcore_barrier()

    e0 = wid * _EPW

    def _chunk(i, _):
        base = e0 + i * _CH
        pltpu.sync_copy(src_hbm.at[pl.ds(base, _CH)], idxs_v)
        pltpu.sync_copy(dst_hbm.at[pl.ds(base, _CH)], idxd_v)
        pltpu.sync_copy(emsg_hbm.at[pl.ds(base, _CH)], emsg_v)
        pltpu.async_copy(x_hbm.at[idxs_v], xrow_v, sem).wait()

        def _row(r, _):
            for jj in range(_D // 16):
                sl = pl.ds(jj * 16, 16)
                v = xrow_v[r, sl] + emsg_v[r, sl]
                xrow_v[r, sl] = jnp.maximum(v, 0.0)
            return 0
        lax.fori_loop(0, _CH, _row, 0)

        pltpu.sync_copy(xrow_v, agg_sh.at[idxd_v], add=True)
        return 0

    lax.fori_loop(0, _NCH, _chunk, 0)
    plsc.subcore_barrier()

    def _wchunk(i, _):
        r = (s + i * _NS) * _ZR
        pltpu.sync_copy(agg_sh.at[pl.ds(r, _ZR)], zbuf_v)
        pltpu.sync_copy(zbuf_v, out_hbm.at[c, pl.ds(r, _ZR)])
        return 0
    lax.fori_loop(0, my_nz, _wchunk, 0)


_sc_aggregate = functools.partial(
    pl.kernel,
    out_type=jax.ShapeDtypeStruct((_NC, _N, _D), jnp.float32),
    mesh=plsc.VectorSubcoreMesh(core_axis_name="c", subcore_axis_name="s"),
    scratch_types=[
        pltpu.VMEM((_CH,), jnp.int32),
        pltpu.VMEM((_CH,), jnp.int32),
        pltpu.VMEM((_CH, _D), jnp.float32),
        pltpu.VMEM((_CH, _D), jnp.float32),
        pltpu.VMEM((_ZR, _D), jnp.float32),
        pltpu.VMEM_SHARED((_N, _D), jnp.float32),
        pltpu.SemaphoreType.DMA,
    ],
)(_sc_body)


_BN = 2000


def _upd_body(s_ref, x_ref, agg_ref, ws_ref, wn_ref, b_ref, out_ref):
    agg = agg_ref[0] + agg_ref[1]
    y = (jnp.dot(x_ref[...], ws_ref[...], preferred_element_type=jnp.float32)
         + jnp.dot(agg, wn_ref[...], preferred_element_type=jnp.float32)
         + b_ref[...])
    out_ref[...] = y * s_ref[0, 0]


def _node_update(x, parts, W_self, W_nbr, b2d, scale):
    return pl.pallas_call(
        _upd_body,
        grid=(_N // _BN,),
        in_specs=[
            pl.BlockSpec(memory_space=pltpu.SMEM),
            pl.BlockSpec((_BN, _D), lambda i: (i, 0)),
            pl.BlockSpec((_NC, _BN, _D), lambda i: (0, i, 0)),
            pl.BlockSpec((_D, _D), lambda i: (0, 0)),
            pl.BlockSpec((_D, _D), lambda i: (0, 0)),
            pl.BlockSpec((1, _D), lambda i: (0, 0)),
        ],
        out_specs=pl.BlockSpec((_BN, _D), lambda i: (i, 0)),
        out_shape=jax.ShapeDtypeStruct((_N, _D), jnp.float32),
    )(scale, x, parts, W_self, W_nbr, b2d)


def kernel(x, edge_index, edge_attr, W_self, W_edge, W_nbr, b, weights_list):
    src = edge_index[0]
    dst = edge_index[1]
    b2d = b.reshape(1, _D)
    sw = jax.nn.softmax(weights_list, axis=-1)[0]

    emsg = _edge_messages(edge_attr, W_edge)
    for i in range(_NUM_BLOCKS):
        parts = _sc_aggregate(x, src, dst, emsg)
        scale = (sw[i, 0] + sw[i, 1]).reshape(1, 1)
        x = _node_update(x, parts, W_self, W_nbr, b2d, scale)
    return x


# speedup vs baseline: 3.9024x; 1.4503x over previous
import functools

import jax
import jax.numpy as jnp
from jax import lax
from jax.experimental import pallas as pl
from jax.experimental.pallas import tpu as pltpu
from jax.experimental.pallas import tpu_sc as plsc

_N = 10000
_E = 320000
_D = 128
_DE = 16
_NUM_BLOCKS = 2

_NC = 2
_NS = 16
_NW = _NC * _NS
_EPW = _E // _NW
_CH = 80
_NCH = _EPW // _CH
_ZR = 80
_NZC = _N // _ZR


_BE = 8000


def _emsg_body(ea_ref, we_ref, out_ref):
    out_ref[...] = jnp.dot(ea_ref[...], we_ref[...],
                           preferred_element_type=jnp.float32)


def _edge_messages(edge_attr, W_edge):
    return pl.pallas_call(
        _emsg_body,
        grid=(_E // _BE,),
        in_specs=[
            pl.BlockSpec((_BE, _DE), lambda i: (i, 0)),
            pl.BlockSpec((_DE, _D), lambda i: (0, 0)),
        ],
        out_specs=pl.BlockSpec((_BE, _D), lambda i: (i, 0)),
        out_shape=jax.ShapeDtypeStruct((_E, _D), jnp.float32),
    )(edge_attr, W_edge)


def _sc_body(x_hbm, src_hbm, dst_hbm, emsg_hbm, out_hbm,
             idxs0, idxs1, idxd0, idxd1, emsg0, emsg1, xrow0, xrow1,
             agg_sh, seml0, seml1, semg0, semg1, sem):
    c = lax.axis_index("c")
    s = lax.axis_index("s")
    wid = c * _NS + s
    idxs = (idxs0, idxs1)
    idxd = (idxd0, idxd1)
    emsg = (emsg0, emsg1)
    xrow = (xrow0, xrow1)
    seml = (seml0, seml1)
    semg = (semg0, semg1)

    def _zfill(i, _):
        r = i // 8
        col = (i % 8) * 16
        emsg0[r, pl.ds(col, 16)] = jnp.zeros((16,), jnp.float32)
        return 0
    lax.fori_loop(0, _ZR * 8, _zfill, 0)
    my_nz = jnp.where(s < _NZC - (_NZC // _NS) * _NS, _NZC // _NS + 1,
                      _NZC // _NS)

    def _zchunk(i, _):
        r = (s + i * _NS) * _ZR
        pltpu.sync_copy(emsg0, agg_sh.at[pl.ds(r, _ZR)])
        return 0
    lax.fori_loop(0, my_nz, _zchunk, 0)
    plsc.subcore_barrier()

    e0 = wid * _EPW

    def _lin_start(i, b):
        base = e0 + i * _CH
        pltpu.async_copy(src_hbm.at[pl.ds(base, _CH)], idxs[b], seml[b])
        pltpu.async_copy(dst_hbm.at[pl.ds(base, _CH)], idxd[b], seml[b])
        pltpu.async_copy(emsg_hbm.at[pl.ds(base, _CH)], emsg[b], seml[b])

    def _lin_wait(i, b):
        base = e0 + i * _CH
        pltpu.make_async_copy(src_hbm.at[pl.ds(base, _CH)], idxs[b], seml[b]).wait()
        pltpu.make_async_copy(dst_hbm.at[pl.ds(base, _CH)], idxd[b], seml[b]).wait()
        pltpu.make_async_copy(emsg_hbm.at[pl.ds(base, _CH)], emsg[b], seml[b]).wait()

    def _gather_start(b):
        pltpu.async_copy(x_hbm.at[idxs[b]], xrow[b], semg[b])

    def _gather_wait(b):
        pltpu.make_async_copy(x_hbm.at[idxs[b]], xrow[b], semg[b]).wait()

    def _compute(b):
        xr = xrow[b]
        em = emsg[b]

        def _row(r, _):
            for jj in range(_D // 16):
                sl = pl.ds(jj * 16, 16)
                xr[r, sl] = jnp.maximum(xr[r, sl] + em[r, sl], 0.0)
            return 0
        lax.fori_loop(0, _CH, _row, 0)

    _lin_start(0, 0)
    _lin_wait(0, 0)
    _gather_start(0)
    _lin_start(1, 1)

    def _step(j, _):
        for b in range(2):
            i = 2 * j + b
            _gather_wait(b)
            _compute(b)
            pltpu.sync_copy(xrow[b], agg_sh.at[idxd[b]], add=True)

            @pl.when(i + 2 < _NCH)
            def _():
                _lin_start(i + 2, b)

            @pl.when(i + 1 < _NCH)
            def _():
                _lin_wait(i + 1, 1 - b)
                _gather_start(1 - b)
        return 0

    lax.fori_loop(0, (_NCH - 1) // 2, _step, 0)

    _gather_wait(0)
    _compute(0)
    pltpu.sync_copy(xrow[0], agg_sh.at[idxd[0]], add=True)

    plsc.subcore_barrier()

    def _wchunk(i, _):
        r = (s + i * _NS) * _ZR
        pltpu.sync_copy(agg_sh.at[pl.ds(r, _ZR)], emsg0)
        pltpu.sync_copy(emsg0, out_hbm.at[c, pl.ds(r, _ZR)])
        return 0
    lax.fori_loop(0, my_nz, _wchunk, 0)


_sc_aggregate = functools.partial(
    pl.kernel,
    out_type=jax.ShapeDtypeStruct((_NC, _N, _D), jnp.float32),
    mesh=plsc.VectorSubcoreMesh(core_axis_name="c", subcore_axis_name="s"),
    scratch_types=[
        pltpu.VMEM((_CH,), jnp.int32),
        pltpu.VMEM((_CH,), jnp.int32),
        pltpu.VMEM((_CH,), jnp.int32),
        pltpu.VMEM((_CH,), jnp.int32),
        pltpu.VMEM((_CH, _D), jnp.float32),
        pltpu.VMEM((_CH, _D), jnp.float32),
        pltpu.VMEM((_CH, _D), jnp.float32),
        pltpu.VMEM((_CH, _D), jnp.float32),
        pltpu.VMEM_SHARED((_N, _D), jnp.float32),
        pltpu.SemaphoreType.DMA,
        pltpu.SemaphoreType.DMA,
        pltpu.SemaphoreType.DMA,
        pltpu.SemaphoreType.DMA,
        pltpu.SemaphoreType.DMA,
    ],
)(_sc_body)


_BN = 2000


def _upd_body(s_ref, x_ref, agg_ref, ws_ref, wn_ref, b_ref, out_ref):
    agg = agg_ref[0] + agg_ref[1]
    y = (jnp.dot(x_ref[...], ws_ref[...], preferred_element_type=jnp.float32)
         + jnp.dot(agg, wn_ref[...], preferred_element_type=jnp.float32)
         + b_ref[...])
    out_ref[...] = y * s_ref[0, 0]


def _node_update(x, parts, W_self, W_nbr, b2d, scale):
    return pl.pallas_call(
        _upd_body,
        grid=(_N // _BN,),
        in_specs=[
            pl.BlockSpec(memory_space=pltpu.SMEM),
            pl.BlockSpec((_BN, _D), lambda i: (i, 0)),
            pl.BlockSpec((_NC, _BN, _D), lambda i: (0, i, 0)),
            pl.BlockSpec((_D, _D), lambda i: (0, 0)),
            pl.BlockSpec((_D, _D), lambda i: (0, 0)),
            pl.BlockSpec((1, _D), lambda i: (0, 0)),
        ],
        out_specs=pl.BlockSpec((_BN, _D), lambda i: (i, 0)),
        out_shape=jax.ShapeDtypeStruct((_N, _D), jnp.float32),
    )(scale, x, parts, W_self, W_nbr, b2d)


def kernel(x, edge_index, edge_attr, W_self, W_edge, W_nbr, b, weights_list):
    src = edge_index[0]
    dst = edge_index[1]
    b2d = b.reshape(1, _D)
    sw = jax.nn.softmax(weights_list, axis=-1)[0]

    emsg = _edge_messages(edge_attr, W_edge)
    for i in range(_NUM_BLOCKS):
        parts = _sc_aggregate(x, src, dst, emsg)
        scale = (sw[i, 0] + sw[i, 1]).reshape(1, 1)
        x = _node_update(x, parts, W_self, W_nbr, b2d, scale)
    return x


# speedup vs baseline: 4.2750x; 1.0955x over previous
import functools

import jax
import jax.numpy as jnp
from jax import lax
from jax.experimental import pallas as pl
from jax.experimental.pallas import tpu as pltpu
from jax.experimental.pallas import tpu_sc as plsc

_N = 10000
_E = 320000
_D = 128
_DE = 16
_NUM_BLOCKS = 2

_NC = 2
_NS = 16
_NW = _NC * _NS
_EPW = _E // _NW
_CH = 80
_NCH = _EPW // _CH
_ZR = 80
_NZC = _N // _ZR


_BE = 8000


def _emsg_body(ea_ref, we_ref, out_ref):
    out_ref[...] = jnp.dot(ea_ref[...], we_ref[...],
                           preferred_element_type=jnp.float32)


def _edge_messages(edge_attr, W_edge):
    return pl.pallas_call(
        _emsg_body,
        grid=(_E // _BE,),
        in_specs=[
            pl.BlockSpec((_BE, _DE), lambda i: (i, 0)),
            pl.BlockSpec((_DE, _D), lambda i: (0, 0)),
        ],
        out_specs=pl.BlockSpec((_BE, _D), lambda i: (i, 0)),
        out_shape=jax.ShapeDtypeStruct((_E, _D), jnp.float32),
    )(edge_attr, W_edge)


def _sc_body(x_hbm, src_hbm, dst_hbm, emsg_hbm, out_hbm,
             idxs0, idxs1, idxd0, idxd1, emsg0, emsg1, xrow0, xrow1,
             agg_sh, seml0, seml1, semg0, semg1, sem):
    c = lax.axis_index("c")
    s = lax.axis_index("s")
    wid = c * _NS + s
    idxs = (idxs0, idxs1)
    idxd = (idxd0, idxd1)
    emsg = (emsg0, emsg1)
    xrow = (xrow0, xrow1)
    seml = (seml0, seml1)
    semg = (semg0, semg1)

    def _zfill(i, _):
        r = i // 8
        col = (i % 8) * 16
        emsg0[r, pl.ds(col, 16)] = jnp.zeros((16,), jnp.float32)
        return 0
    lax.fori_loop(0, _ZR * 8, _zfill, 0)
    my_nz = jnp.where(s < _NZC - (_NZC // _NS) * _NS, _NZC // _NS + 1,
                      _NZC // _NS)

    def _zchunk(i, _):
        r = (s + i * _NS) * _ZR
        pltpu.sync_copy(emsg0, agg_sh.at[pl.ds(r, _ZR)])
        return 0
    lax.fori_loop(0, my_nz, _zchunk, 0)
    plsc.subcore_barrier()

    e0 = wid * _EPW

    def _lin_start(i, b):
        base = e0 + i * _CH
        pltpu.async_copy(src_hbm.at[pl.ds(base, _CH)], idxs[b], seml[b])
        pltpu.async_copy(dst_hbm.at[pl.ds(base, _CH)], idxd[b], seml[b])
        pltpu.async_copy(emsg_hbm.at[pl.ds(base, _CH)], emsg[b], seml[b])

    def _lin_wait(i, b):
        base = e0 + i * _CH
        pltpu.make_async_copy(src_hbm.at[pl.ds(base, _CH)], idxs[b], seml[b]).wait()
        pltpu.make_async_copy(dst_hbm.at[pl.ds(base, _CH)], idxd[b], seml[b]).wait()
        pltpu.make_async_copy(emsg_hbm.at[pl.ds(base, _CH)], emsg[b], seml[b]).wait()

    def _gather_start(b):
        pltpu.async_copy(x_hbm.at[idxs[b]], xrow[b], semg[b])

    def _gather_wait(b):
        pltpu.make_async_copy(x_hbm.at[idxs[b]], xrow[b], semg[b]).wait()

    def _compute(b):
        xr = xrow[b]
        em = emsg[b]

        def _row(r, _):
            for jj in range(_D // 16):
                sl = pl.ds(jj * 16, 16)
                xr[r, sl] = jnp.maximum(xr[r, sl] + em[r, sl], 0.0)
            return 0
        lax.fori_loop(0, _CH, _row, 0)

    _lin_start(0, 0)
    _lin_wait(0, 0)
    _gather_start(0)
    _lin_start(1, 1)

    def _step(j, _):
        for b in range(2):
            i = 2 * j + b
            _gather_wait(b)

            @pl.when(i + 1 < _NCH)
            def _():
                _lin_wait(i + 1, 1 - b)
                _gather_start(1 - b)

            _compute(b)
            pltpu.sync_copy(xrow[b], agg_sh.at[idxd[b]], add=True)

            @pl.when(i + 2 < _NCH)
            def _():
                _lin_start(i + 2, b)
        return 0

    lax.fori_loop(0, (_NCH - 1) // 2, _step, 0)

    _gather_wait(0)
    _compute(0)
    pltpu.sync_copy(xrow[0], agg_sh.at[idxd[0]], add=True)

    plsc.subcore_barrier()

    def _wchunk(i, _):
        r = (s + i * _NS) * _ZR
        pltpu.sync_copy(agg_sh.at[pl.ds(r, _ZR)], emsg0)
        pltpu.sync_copy(emsg0, out_hbm.at[c, pl.ds(r, _ZR)])
        return 0
    lax.fori_loop(0, my_nz, _wchunk, 0)


_sc_aggregate = functools.partial(
    pl.kernel,
    out_type=jax.ShapeDtypeStruct((_NC, _N, _D), jnp.float32),
    mesh=plsc.VectorSubcoreMesh(core_axis_name="c", subcore_axis_name="s"),
    scratch_types=[
        pltpu.VMEM((_CH,), jnp.int32),
        pltpu.VMEM((_CH,), jnp.int32),
        pltpu.VMEM((_CH,), jnp.int32),
        pltpu.VMEM((_CH,), jnp.int32),
        pltpu.VMEM((_CH, _D), jnp.float32),
        pltpu.VMEM((_CH, _D), jnp.float32),
        pltpu.VMEM((_CH, _D), jnp.float32),
        pltpu.VMEM((_CH, _D), jnp.float32),
        pltpu.VMEM_SHARED((_N, _D), jnp.float32),
        pltpu.SemaphoreType.DMA,
        pltpu.SemaphoreType.DMA,
        pltpu.SemaphoreType.DMA,
        pltpu.SemaphoreType.DMA,
        pltpu.SemaphoreType.DMA,
    ],
)(_sc_body)


_BN = 2000


def _upd_body(s_ref, x_ref, agg_ref, ws_ref, wn_ref, b_ref, out_ref):
    agg = agg_ref[0] + agg_ref[1]
    y = (jnp.dot(x_ref[...], ws_ref[...], preferred_element_type=jnp.float32)
         + jnp.dot(agg, wn_ref[...], preferred_element_type=jnp.float32)
         + b_ref[...])
    out_ref[...] = y * s_ref[0, 0]


def _node_update(x, parts, W_self, W_nbr, b2d, scale):
    return pl.pallas_call(
        _upd_body,
        grid=(_N // _BN,),
        in_specs=[
            pl.BlockSpec(memory_space=pltpu.SMEM),
            pl.BlockSpec((_BN, _D), lambda i: (i, 0)),
            pl.BlockSpec((_NC, _BN, _D), lambda i: (0, i, 0)),
            pl.BlockSpec((_D, _D), lambda i: (0, 0)),
            pl.BlockSpec((_D, _D), lambda i: (0, 0)),
            pl.BlockSpec((1, _D), lambda i: (0, 0)),
        ],
        out_specs=pl.BlockSpec((_BN, _D), lambda i: (i, 0)),
        out_shape=jax.ShapeDtypeStruct((_N, _D), jnp.float32),
    )(scale, x, parts, W_self, W_nbr, b2d)


def kernel(x, edge_index, edge_attr, W_self, W_edge, W_nbr, b, weights_list):
    src = edge_index[0]
    dst = edge_index[1]
    b2d = b.reshape(1, _D)
    sw = jax.nn.softmax(weights_list, axis=-1)[0]

    emsg = _edge_messages(edge_attr, W_edge)
    for i in range(_NUM_BLOCKS):
        parts = _sc_aggregate(x, src, dst, emsg)
        scale = (sw[i, 0] + sw[i, 1]).reshape(1, 1)
        x = _node_update(x, parts, W_self, W_nbr, b2d, scale)
    return x


# speedup vs baseline: 5.5418x; 1.2963x over previous
import functools

import jax
import jax.numpy as jnp
from jax import lax
from jax.experimental import pallas as pl
from jax.experimental.pallas import tpu as pltpu
from jax.experimental.pallas import tpu_sc as plsc

_N = 10000
_E = 320000
_D = 128
_DE = 16
_NUM_BLOCKS = 2

_NC = 2
_NS = 16
_NW = _NC * _NS
_EPW = _E // _NW
_CH = 80
_NCH = _EPW // _CH
_ZR = 80
_NZC = _N // _ZR


_BE = 8000


def _emsg_body(ea_ref, we_ref, out_ref):
    out_ref[...] = jnp.dot(ea_ref[...], we_ref[...],
                           preferred_element_type=jnp.float32)


def _edge_messages(edge_attr, W_edge):
    return pl.pallas_call(
        _emsg_body,
        grid=(_E // _BE,),
        in_specs=[
            pl.BlockSpec((_BE, _DE), lambda i: (i, 0)),
            pl.BlockSpec((_DE, _D), lambda i: (0, 0)),
        ],
        out_specs=pl.BlockSpec((_BE, _D), lambda i: (i, 0)),
        out_shape=jax.ShapeDtypeStruct((_E, _D), jnp.float32),
    )(edge_attr, W_edge)


def _sc_body(x_hbm, src_hbm, dst_hbm, emsg_hbm, out_hbm,
             idxs0, idxs1, idxd0, idxd1, idxq0, idxq1,
             emsg0, emsg1, xrow0, xrow1,
             agg_sh, seml0, seml1, semg0, semg1, sems0, sems1, sem):
    c = lax.axis_index("c")
    s = lax.axis_index("s")
    wid = c * _NS + s
    idxs = (idxs0, idxs1)
    idxd = (idxd0, idxd1)
    idxq = (idxq0, idxq1)
    emsg = (emsg0, emsg1)
    xrow = (xrow0, xrow1)
    seml = (seml0, seml1)
    semg = (semg0, semg1)
    sems = (sems0, sems1)

    def _zfill(i, _):
        r = i // 8
        col = (i % 8) * 16
        emsg0[r, pl.ds(col, 16)] = jnp.zeros((16,), jnp.float32)
        return 0
    lax.fori_loop(0, _ZR * 8, _zfill, 0)
    my_nz = jnp.where(s < _NZC - (_NZC // _NS) * _NS, _NZC // _NS + 1,
                      _NZC // _NS)

    def _zchunk(i, _):
        r = (s + i * _NS) * _ZR
        pltpu.sync_copy(emsg0, agg_sh.at[pl.ds(r, _ZR)])
        return 0
    lax.fori_loop(0, my_nz, _zchunk, 0)
    plsc.subcore_barrier()

    e0 = wid * _EPW

    def _lin_start(i, b):
        base = e0 + i * _CH
        pltpu.async_copy(src_hbm.at[pl.ds(base, _CH)], idxs[b], seml[b])
        pltpu.async_copy(dst_hbm.at[pl.ds(base, _CH)], idxd[b], seml[b])
        pltpu.async_copy(emsg_hbm.at[pl.ds(base, _CH)], emsg[b], seml[b])

    def _lin_wait(i, b):
        base = e0 + i * _CH
        pltpu.make_async_copy(src_hbm.at[pl.ds(base, _CH)], idxs[b], seml[b]).wait()
        pltpu.make_async_copy(dst_hbm.at[pl.ds(base, _CH)], idxd[b], seml[b]).wait()
        pltpu.make_async_copy(emsg_hbm.at[pl.ds(base, _CH)], emsg[b], seml[b]).wait()

    def _gather_start(b):
        pltpu.async_copy(x_hbm.at[idxs[b]], xrow[b], semg[b])

    def _gather_wait(b):
        pltpu.make_async_copy(x_hbm.at[idxs[b]], xrow[b], semg[b]).wait()

    def _compute(b):
        xr = xrow[b]
        em = emsg[b]

        @functools.partial(plsc.parallel_loop, 0, _CH, unroll=4)
        def _row(r):
            for jj in range(_D // 16):
                sl = pl.ds(jj * 16, 16)
                xr[r, sl] = jnp.maximum(xr[r, sl] + em[r, sl], 0.0)

    def _scat_start(b):
        for k in range(_CH // 16):
            sl = pl.ds(k * 16, 16)
            idxq[b][sl] = idxd[b][sl]
        pltpu.async_copy(xrow[b], agg_sh.at[idxq[b]], sems[b], add=True)

    def _scat_wait(b):
        pltpu.make_async_copy(xrow[b], agg_sh.at[idxq[b]], sems[b]).wait()

    _lin_start(0, 0)
    _lin_wait(0, 0)
    _gather_start(0)
    _lin_start(1, 1)

    def _step(j, _):
        for b in range(2):
            i = 2 * j + b
            _gather_wait(b)

            @pl.when(i > 0)
            def _():
                _scat_wait(1 - b)

            @pl.when(i + 1 < _NCH)
            def _():
                _lin_wait(i + 1, 1 - b)
                _gather_start(1 - b)

            _compute(b)
            _scat_start(b)

            @pl.when(i + 2 < _NCH)
            def _():
                _lin_start(i + 2, b)
        return 0

    lax.fori_loop(0, (_NCH - 1) // 2, _step, 0)

    _gather_wait(0)
    _scat_wait(1)
    _compute(0)
    _scat_start(0)
    _scat_wait(0)

    plsc.subcore_barrier()

    def _wchunk(i, _):
        r = (s + i * _NS) * _ZR
        pltpu.sync_copy(agg_sh.at[pl.ds(r, _ZR)], emsg0)
        pltpu.sync_copy(emsg0, out_hbm.at[c, pl.ds(r, _ZR)])
        return 0
    lax.fori_loop(0, my_nz, _wchunk, 0)


_sc_aggregate = functools.partial(
    pl.kernel,
    out_type=jax.ShapeDtypeStruct((_NC, _N, _D), jnp.float32),
    mesh=plsc.VectorSubcoreMesh(core_axis_name="c", subcore_axis_name="s"),
    scratch_types=[
        pltpu.VMEM((_CH,), jnp.int32),
        pltpu.VMEM((_CH,), jnp.int32),
        pltpu.VMEM((_CH,), jnp.int32),
        pltpu.VMEM((_CH,), jnp.int32),
        pltpu.VMEM((_CH,), jnp.int32),
        pltpu.VMEM((_CH,), jnp.int32),
        pltpu.VMEM((_CH, _D), jnp.float32),
        pltpu.VMEM((_CH, _D), jnp.float32),
        pltpu.VMEM((_CH, _D), jnp.float32),
        pltpu.VMEM((_CH, _D), jnp.float32),
        pltpu.VMEM_SHARED((_N, _D), jnp.float32),
        pltpu.SemaphoreType.DMA,
        pltpu.SemaphoreType.DMA,
        pltpu.SemaphoreType.DMA,
        pltpu.SemaphoreType.DMA,
        pltpu.SemaphoreType.DMA,
        pltpu.SemaphoreType.DMA,
        pltpu.SemaphoreType.DMA,
    ],
)(_sc_body)


_BN = 2000


def _upd_body(s_ref, x_ref, agg_ref, ws_ref, wn_ref, b_ref, out_ref):
    agg = agg_ref[0] + agg_ref[1]
    y = (jnp.dot(x_ref[...], ws_ref[...], preferred_element_type=jnp.float32)
         + jnp.dot(agg, wn_ref[...], preferred_element_type=jnp.float32)
         + b_ref[...])
    out_ref[...] = y * s_ref[0, 0]


def _node_update(x, parts, W_self, W_nbr, b2d, scale):
    return pl.pallas_call(
        _upd_body,
        grid=(_N // _BN,),
        in_specs=[
            pl.BlockSpec(memory_space=pltpu.SMEM),
            pl.BlockSpec((_BN, _D), lambda i: (i, 0)),
            pl.BlockSpec((_NC, _BN, _D), lambda i: (0, i, 0)),
            pl.BlockSpec((_D, _D), lambda i: (0, 0)),
            pl.BlockSpec((_D, _D), lambda i: (0, 0)),
            pl.BlockSpec((1, _D), lambda i: (0, 0)),
        ],
        out_specs=pl.BlockSpec((_BN, _D), lambda i: (i, 0)),
        out_shape=jax.ShapeDtypeStruct((_N, _D), jnp.float32),
    )(scale, x, parts, W_self, W_nbr, b2d)


def kernel(x, edge_index, edge_attr, W_self, W_edge, W_nbr, b, weights_list):
    src = edge_index[0]
    dst = edge_index[1]
    b2d = b.reshape(1, _D)
    sw = jax.nn.softmax(weights_list, axis=-1)[0]

    emsg = _edge_messages(edge_attr, W_edge)
    for i in range(_NUM_BLOCKS):
        parts = _sc_aggregate(x, src, dst, emsg)
        scale = (sw[i, 0] + sw[i, 1]).reshape(1, 1)
        x = _node_update(x, parts, W_self, W_nbr, b2d, scale)
    return x
